# baseline (speedup 1.0000x reference)
import functools
import jax
import jax.numpy as jnp
from jax import lax
from jax.experimental import pallas as pl
from jax.experimental.pallas import tpu as pltpu
from jax.experimental.pallas import tpu_sc as plsc

N = 10000
D_INF = 128
MUL = 32
MAX_RADIUS = 0.073
E_MAX = 262144
BLK = 2048
STEP = MAX_RADIUS / 11.0
CEMB = 1.14136 * 2.718281828459045 ** 2
N2 = 1.0 / (MUL * 2.0) ** 0.5
N3 = 1.0 / (MUL * 3.0) ** 0.5
SSC = 1.0 / 15.0 ** 0.5
SQ3 = 3.0 ** 0.5
I_SQ3 = 1.0 / 3.0 ** 0.5
I_SQ2 = 1.0 / 2.0 ** 0.5

NW = 32
RPW = 313
NPAD = NW * RPW
K = 64

RPAD = 10240
CPAD = 10240
BRA = 256
E_CAP = 264192
FLUSH = 512
RING = 592

_DIMS = {0: (64, 128), 1: (352, 224), 2: (320, 128)}
_DPAD = {0: 128, 1: 384, 2: 384}


def _mask_body(pb_ref, pt_ref, m_ref, c_ref):
    pb = pb_ref[...]
    pt = pt_ref[...]
    dot = jnp.dot(pb, pt, preferred_element_type=jnp.float32)
    sqr = jnp.sum(pb * pb, axis=1, keepdims=True)
    sqc = jnp.sum(pt * pt, axis=0, keepdims=True)
    d2 = sqr + sqc - 2.0 * dot
    colv = jax.lax.broadcasted_iota(jnp.int32, (BRA, CPAD), 1)
    rowv = (jax.lax.broadcasted_iota(jnp.int32, (BRA, CPAD), 0)
            + pl.program_id(0) * BRA)
    m = jnp.logical_and(d2 <= MAX_RADIUS ** 2, colv != rowv)
    m_ref[...] = m.astype(jnp.int8)
    c_ref[...] = jnp.sum(m.astype(jnp.int32), axis=1, keepdims=True)


@functools.lru_cache(maxsize=None)
def _mask_kernel():
    return pl.pallas_call(
        _mask_body,
        grid=(RPAD // BRA,),
        in_specs=[
            pl.BlockSpec((BRA, 8), lambda i: (i, 0)),
            pl.BlockSpec((8, CPAD), lambda i: (0, 0)),
        ],
        out_specs=[
            pl.BlockSpec((BRA, CPAD), lambda i: (i, 0)),
            pl.BlockSpec((BRA, 1), lambda i: (i, 0)),
        ],
        out_shape=[
            jax.ShapeDtypeStruct((RPAD, CPAD), jnp.int8),
            jax.ShapeDtypeStruct((RPAD, 1), jnp.int32),
        ],
    )


_IOTA4 = None


def _compact_body(mask_hbm, bw_hbm, col_out, row_out, bw_v, rb_v, cr_v,
                  rr_v, sem):
    info = plsc.get_sparse_core_info()
    nc = info.num_cores
    wid = lax.axis_index("s") * nc + lax.axis_index("c")
    pltpu.sync_copy(bw_hbm, bw_v)
    bw = bw_v[pl.ds(4 * wid, 16)]
    gpos0, rlo, rhi, padv = bw[0], bw[1], bw[2], bw[3]
    iota4 = lax.iota(jnp.int32, 16) * 4
    lanei = lax.iota(jnp.int32, 16)

    def hsum(x):
        for sh in (8, 4, 2, 1):
            idx = jnp.bitwise_and(lanei + sh, 15)
            x = x + x.at[idx].get(mode='promise_in_bounds')
        return x[0]

    def row_loop(i, carry):
        off = pl.multiple_of(i * (CPAD // 4), 512)
        pltpu.sync_copy(mask_hbm.at[pl.ds(off, CPAD // 4)], rb_v)

        def chunk(k, c2):
            v = rb_v[pl.ds(k * 16, 16)]
            anym = hsum(jnp.where(v != 0, 1, 0)) > 0

            def hit(c3):
                fill, gpos = c3
                cbase = k * 64

                for b in range(4):
                    vb = jnp.bitwise_and(
                        jnp.right_shift(v, 8 * b), 255)

                    def dosub(f, vb=vb, b=b):
                        for lane in range(16):
                            cs = cbase + 4 * lane + b
                            w = cr_v[pl.ds(f, 16)]
                            cr_v[pl.ds(f, 16)] = jnp.where(
                                lanei == 0, cs, w)
                            w2 = rr_v[pl.ds(f, 16)]
                            rr_v[pl.ds(f, 16)] = jnp.where(
                                lanei == 0, i, w2)
                            f = f + vb[lane]
                        return f

                    fill = lax.cond(hsum(vb) > 0, dosub,
                                    lambda f: f, fill)

                def doflush(c4):
                    f2, g2 = c4
                    ga = pl.multiple_of(g2, 8)
                    pltpu.sync_copy(cr_v.at[pl.ds(0, FLUSH)],
                                    col_out.at[pl.ds(ga, FLUSH)])
                    pltpu.sync_copy(rr_v.at[pl.ds(0, FLUSH)],
                                    row_out.at[pl.ds(ga, FLUSH)])
                    for t in range(5):
                        cr_v[pl.ds(t * 16, 16)] = cr_v[
                            pl.ds(FLUSH + t * 16, 16)]
                        rr_v[pl.ds(t * 16, 16)] = rr_v[
                            pl.ds(FLUSH + t * 16, 16)]
                    return f2 - FLUSH, g2 + FLUSH

                do = jnp.logical_and(fill >= FLUSH,
                                     gpos + FLUSH <= E_CAP)
                return lax.cond(do, doflush, lambda c4: c4, (fill, gpos))

            return lax.cond(anym, hit, lambda c3: c3, c2)

        return lax.fori_loop(0, CPAD // 64, chunk, carry)

    fill, gpos = lax.fori_loop(rlo, rhi, row_loop, (gpos0 * 0, gpos0))
    fb = (fill // 16) * 16
    for ring in (cr_v, rr_v):
        win = ring[pl.ds(fb, 16)]
        ring[pl.ds(fb, 16)] = jnp.where(lanei < fill - fb, win, padv)
    nt = (fill + 15) // 16

    def tail(t, g2):
        ga = pl.multiple_of(g2, 8)

        @pl.when(ga + 16 <= E_CAP)
        def _():
            pltpu.sync_copy(cr_v.at[pl.ds(t * 16, 16)],
                            col_out.at[pl.ds(ga, 16)])
            pltpu.sync_copy(rr_v.at[pl.ds(t * 16, 16)],
                            row_out.at[pl.ds(ga, 16)])
        return g2 + 16

    lax.fori_loop(0, nt, tail, gpos)


@functools.lru_cache(maxsize=None)
def _compact_kernel():
    mesh = plsc.VectorSubcoreMesh(core_axis_name="c", subcore_axis_name="s")
    return pl.kernel(
        _compact_body,
        mesh=mesh,
        out_type=[
            jax.ShapeDtypeStruct((E_CAP,), jnp.int32),
            jax.ShapeDtypeStruct((E_CAP,), jnp.int32),
        ],
        scratch_types=[
            pltpu.VMEM((4 * NW + 16,), jnp.int32),
            pltpu.VMEM((CPAD // 4,), jnp.int32),
            pltpu.VMEM((RING,), jnp.int32),
            pltpu.VMEM((RING,), jnp.int32),
            pltpu.SemaphoreType.DMA,
        ],
    )


def _geom_body(g_ref, fc1_ref, fc2_ref, o_ref):
    g = g_ref[...]
    evx, evy, evz = g[:, 0:1], g[:, 1:2], g[:, 2:3]
    valid = g[:, 3:4]
    el2 = evx * evx + evy * evy + evz * evz
    el = jnp.sqrt(el2)
    inv = jnp.where(el > 0.0, 1.0 / jnp.where(el > 0.0, el, 1.0), 0.0)
    shx, shy, shz = SQ3 * evx * inv, SQ3 * evy * inv, SQ3 * evz * inv
    lane = jax.lax.broadcasted_iota(
        jnp.int32, (el.shape[0], 16), 1).astype(jnp.float32)
    diff = (el - (lane + 1.0) * STEP) * (1.0 / STEP)
    dd = diff * diff
    inside = jnp.logical_and(dd < 1.0, lane < 10.0)
    safe = jnp.where(inside, dd, 0.0)
    emb = jnp.where(inside, CEMB * jnp.exp(-1.0 / (1.0 - safe)), 0.0)
    rs = []
    for l in range(3):
        hid = jnp.dot(emb, fc1_ref[16 * l:16 * (l + 1), :],
                      preferred_element_type=jnp.float32)
        hid = hid * jax.nn.sigmoid(hid)
        rs.append(jnp.sum(hid * fc2_ref[8 * l:8 * l + 1, :], axis=1,
                          keepdims=True) * 0.125 * valid)
    o_ref[...] = jnp.concatenate(
        [shx, shy, shz, rs[0], rs[1], rs[2],
         jnp.zeros_like(shx), jnp.zeros_like(shx)], axis=1)


@functools.lru_cache(maxsize=None)
def _geom_kernel():
    return pl.pallas_call(
        _geom_body,
        grid=(E_CAP // BLK,),
        in_specs=[
            pl.BlockSpec((BLK, 8), lambda i: (i, 0)),
            pl.BlockSpec((48, 64), lambda i: (0, 0)),
            pl.BlockSpec((24, 64), lambda i: (0, 0)),
        ],
        out_specs=pl.BlockSpec((BLK, 8), lambda i: (i, 0)),
        out_shape=jax.ShapeDtypeStruct((E_CAP, 8), jnp.float32),
    )


def _sc_layer_body(layer, t_hbm, col_hbm, row_hbm, g_hbm, b_hbm, out_hbm,
                   b_v, idx_v, rows_v, g_v, row_v, acc_v, sem):
    d_in, f_out = _DIMS[layer]
    info = plsc.get_sparse_core_info()
    nc = info.num_cores
    wid = lax.axis_index("s") * nc + lax.axis_index("c")
    pltpu.sync_copy(b_hbm, b_v)
    row0 = wid * RPW
    bb = b_v[pl.ds(2 * wid, 16)]

    zeros16 = jnp.zeros((16,), jnp.float32)

    def zbody(i, _):
        acc_v[pl.ds(i * 16, 16)] = zeros16
        return 0

    lax.fori_loop(0, RPW * f_out // 16, zbody, 0)

    e0 = bb[0]
    e1 = bb[1]
    nchunk = (e1 - e0 + (K - 1)) // K

    def chunk(c, _):
        ea = pl.multiple_of(e0 + c * K, 8)
        pltpu.sync_copy(col_hbm.at[pl.ds(ea, K)], idx_v)
        pltpu.sync_copy(row_hbm.at[pl.ds(ea, K)], row_v.at[pl.ds(0, K)])
        pltpu.sync_copy(g_hbm.at[pl.ds(ea * 8, K * 8)], g_v.at[pl.ds(0, K * 8)])
        pltpu.async_copy(t_hbm.at[idx_v], rows_v, sem).wait()

        def edge(j, _):
            ge = ea + j
            lr = row_v[pl.ds(j, 16)][0] - row0
            ok = jnp.logical_and(
                ge < e1,
                jnp.logical_and(lr >= 0, lr < RPW))

            @pl.when(ok)
            def _():
                gv = g_v[pl.ds(j * 8, 16)]
                shx = gv[0]
                shy = gv[1]
                shz = gv[2]
                rr = gv[3 + layer]
                r2 = rr * N2
                r3 = rr * N3
                ab = lr * f_out

                def ld(t, h):
                    return rows_v[j, pl.ds(32 * t + 16 * h, 16)]

                def st(t, h, v):
                    plsc.addupdate(acc_v.at[pl.ds(ab + 32 * t + 16 * h, 16)], v)

                for h in range(2):
                    if layer == 0:
                        t2, t3 = ld(0, h), ld(1, h)
                        st(0, h, t2 * r2)
                        st(1, h, t3 * (shx * r3))
                        st(2, h, t3 * (shy * r3))
                        st(3, h, t3 * (shz * r3))
                    elif layer == 1:
                        u2, u3 = ld(0, h), ld(1, h)
                        u4x, u4y, u4z = ld(2, h), ld(3, h), ld(4, h)
                        u5x, u5y, u5z = ld(5, h), ld(6, h), ld(7, h)
                        u6x, u6y, u6z = ld(8, h), ld(9, h), ld(10, h)
                        dot5 = (u5x * shx + u5y * shy + u5z * shz) * I_SQ3
                        st(0, h, (u2 + dot5) * r2)
                        st(1, h, (u3 * shx + u4x) * r3)
                        st(2, h, (u3 * shy + u4y) * r3)
                        st(3, h, (u3 * shz + u4z) * r3)
                        cs = I_SQ2 * r3
                        st(4, h, (u6y * shz - u6z * shy) * cs)
                        st(5, h, (u6z * shx - u6x * shz) * cs)
                        st(6, h, (u6x * shy - u6y * shx) * cs)
                    else:
                        v2 = ld(0, h)
                        v5x, v5y, v5z = ld(1, h), ld(2, h), ld(3, h)
                        v6x, v6y, v6z = ld(4, h), ld(5, h), ld(6, h)
                        v7x, v7y, v7z = ld(7, h), ld(8, h), ld(9, h)
                        dot5 = (v5x * shx + v5y * shy + v5z * shz) * I_SQ3
                        st(0, h, (v2 + dot5) * r2)
                        st(1, h, (v7x + (v6y * shz - v6z * shy) * I_SQ2) * r3)
                        st(2, h, (v7y + (v6z * shx - v6x * shz) * I_SQ2) * r3)
                        st(3, h, (v7z + (v6x * shy - v6y * shx) * I_SQ2) * r3)
            return 0

        lax.fori_loop(0, K, edge, 0)
        return 0

    lax.fori_loop(0, nchunk, chunk, 0)
    pltpu.sync_copy(acc_v, out_hbm.at[pl.ds(wid * RPW * f_out, RPW * f_out)])


@functools.lru_cache(maxsize=None)
def _sc_layer_kernel(layer):
    d_in, f_out = _DIMS[layer]
    mesh = plsc.VectorSubcoreMesh(core_axis_name="c", subcore_axis_name="s")
    return pl.kernel(
        functools.partial(_sc_layer_body, layer),
        mesh=mesh,
        out_type=jax.ShapeDtypeStruct((NPAD * f_out,), jnp.float32),
        scratch_types=[
            pltpu.VMEM((2 * NW + 16,), jnp.int32),
            pltpu.VMEM((K,), jnp.int32),
            pltpu.VMEM((K, _DPAD[layer]), jnp.float32),
            pltpu.VMEM((K * 8 + 16,), jnp.float32),
            pltpu.VMEM((K + 16,), jnp.int32),
            pltpu.VMEM((RPW * f_out,), jnp.float32),
            pltpu.SemaphoreType.DMA,
        ],
    )


def _mulmat(h, w):
    return jnp.einsum('ncu,uv->ncv', h, w).reshape(N, 96)


def kernel(pos, x, orientation, w_emb, tp_w, fc_w1, fc_w2, wd0, wd1):
    sq = jnp.sum(pos * pos, axis=1)
    d2 = sq[:, None] + sq[None, :] - 2.0 * (pos @ pos.T)
    ii = jnp.arange(N, dtype=jnp.int32)
    m = jnp.logical_and(d2 <= MAX_RADIUS ** 2, ii[:, None] != ii[None, :])
    maskb = jnp.zeros((RPAD, CPAD), jnp.int8).at[:N, :N].set(
        m.astype(jnp.int8))
    counts = jnp.sum(m, axis=1).astype(jnp.int32)

    cnt_w = jnp.sum(
        jnp.concatenate([counts, jnp.zeros((NPAD - N,), jnp.int32)])
        .reshape(NW, RPW), axis=1)
    cap_w = ((cnt_w + 15) // 16) * 16
    astart = jnp.minimum(
        jnp.concatenate([jnp.zeros((1,), jnp.int32),
                         jnp.cumsum(cap_w)]).astype(jnp.int32),
        E_CAP - 16)
    rlo = jnp.arange(NW, dtype=jnp.int32) * RPW
    rhi = jnp.minimum(rlo + RPW, N)
    padv = jnp.minimum(rlo + RPW - 1, N - 1)
    bw = jnp.concatenate(
        [jnp.stack([astart[:NW], rlo, rhi, padv], axis=1).reshape(-1),
         jnp.zeros((16,), jnp.int32)])

    mask32 = jax.lax.bitcast_convert_type(
        maskb.reshape(RPAD * CPAD // 4, 4), jnp.int32)
    col, row_s = _compact_kernel()(mask32, bw)
    col = jnp.clip(col, 0, N - 1)
    row_c = jnp.clip(row_s, 0, N - 1)
    ev = pos[col] - pos[row_c]
    valid = jnp.sum(ev * ev, axis=1) > 0.0
    G = jnp.concatenate(
        [ev, valid.astype(jnp.float32)[:, None],
         jnp.zeros((E_CAP, 4), jnp.float32)], axis=1)

    bounds = jnp.concatenate(
        [jnp.stack([astart[:NW], astart[1:]], axis=1).reshape(64),
         jnp.zeros((16,), jnp.int32)])

    fc1p = jnp.zeros((3, 16, 64), jnp.float32).at[:, :10, :].set(fc_w1)
    fc2p = jnp.zeros((3, 8, 64), jnp.float32).at[:, 0, :].set(fc_w2[:, :, 0])
    G2 = _geom_kernel()(G, fc1p.reshape(48, 64), fc2p.reshape(24, 64))

    h0e = x @ w_emb * (1.0 / float(D_INF) ** 0.5)

    def run_layer(l, T):
        Tp = jnp.concatenate(
            [T, jnp.zeros((N, _DPAD[l] - T.shape[1]), jnp.float32)], axis=1)
        flat = _sc_layer_kernel(l)(Tp, col, row_s, G2.reshape(-1), bounds)
        return flat.reshape(NPAD, _DIMS[l][1])[:N] * SSC

    W = tp_w[0]
    d = run_layer(0, jnp.concatenate([h0e @ W[2], h0e @ W[3]], axis=1))
    h0e = h0e + d[:, :32]
    h1o = d[:, 32:].reshape(N, 3, MUL)

    W = tp_w[1]
    d = run_layer(1, jnp.concatenate(
        [h0e @ W[2], h0e @ W[3], _mulmat(h1o, W[4]), _mulmat(h1o, W[5]),
         _mulmat(h1o, W[6])], axis=1))
    h0e = h0e + d[:, :32]
    h1o = h1o + d[:, 32:128].reshape(N, 3, MUL)
    h1e = d[:, 128:].reshape(N, 3, MUL)

    W = tp_w[2]
    d = run_layer(2, jnp.concatenate(
        [h0e @ W[2], _mulmat(h1o, W[5]), _mulmat(h1o, W[6]),
         _mulmat(h1e, W[7])], axis=1))
    h0e = h0e + d[:, :32]
    h1e = h1e + d[:, 32:].reshape(N, 3, MUL)

    c0 = jnp.mean(h0e, axis=0) @ wd0 * (1.0 / float(MUL) ** 0.5)
    c1 = jnp.mean(h1e, axis=0) @ wd1 * (1.0 / float(MUL) ** 0.5)
    sh_coeffs = jnp.concatenate([c0[None], c1])[None, :]
    theta, phi = orientation[..., 0], orientation[..., 1]
    v = jnp.stack([jnp.sin(theta) * jnp.cos(phi),
                   jnp.sin(theta) * jnp.sin(phi),
                   jnp.cos(theta)], axis=-1)
    sh_q = jnp.concatenate([jnp.ones_like(theta)[..., None], SQ3 * v],
                           axis=-1)
    return jnp.sum(sh_coeffs * sh_q, axis=-1)
